# Initial kernel scaffold; baseline (speedup 1.0000x reference)
import jax
import jax.numpy as jnp
from jax.experimental import pallas as pl


def kernel(feat, edge_index, edge_w, Q_w, Q_b, W_w, W_b):
    raise NotImplementedError("write your pallas kernel here")


# speedup vs baseline: 4.6724x; 4.6724x over previous
import functools

import jax
import jax.numpy as jnp
from jax import lax
from jax.experimental import pallas as pl
from jax.experimental.pallas import tpu as pltpu
from jax.experimental.pallas import tpu_sc as plsc

N = 10000
E = 320000
D = 128
NC = 2
NS = 16
NW = NC * NS
EPW = E // NW
K = 80
NCHUNK = EPW // K
RPS = 624
RTAIL = N - RPS * NS


def _mm1_body(x_ref, w_ref, b_ref, o_ref):
    o_ref[...] = jnp.maximum(
        jnp.dot(x_ref[...], w_ref[...], preferred_element_type=jnp.float32)
        + b_ref[...], 0.0)


def _post_body(feat_ref, num_ref, den_ref, deg_ref, w1_ref, w2_ref, b_ref,
               o_ref):
    num = num_ref[0] + num_ref[1]
    den = jnp.sum(den_ref[...], axis=0)[:, None]
    deg = jnp.sum(deg_ref[...], axis=0)[:, None]
    safe_den = jnp.where(den > 0, den, 1.0)
    agg = jnp.where(den > 0, num / safe_den, 0.0)
    rst = jnp.dot(feat_ref[...], w1_ref[...], preferred_element_type=jnp.float32)
    rst = rst + jnp.dot(agg, w2_ref[...], preferred_element_type=jnp.float32)
    rst = jnp.maximum(rst + b_ref[...], 0.0)
    rst = rst * (1.0 / jnp.maximum(deg, 1.0))
    denom = jnp.sqrt(jnp.sum(rst * rst, axis=1, keepdims=True))
    safe_denom = jnp.where(denom > 0, denom, 1.0)
    o_ref[...] = jnp.where(denom > 0, rst / safe_denom, 0.0)


def _sc_body(h_hbm, src_hbm, dst_hbm, ew_hbm, z128_hbm,
             num_out, den_out, deg_out,
             src_v, dst_v, ew_v, rows_v, den_v, deg_v, acc_sh, sem):
    cid = lax.axis_index("c")
    sid = lax.axis_index("s")
    wid = cid * NS + sid

    row0 = sid * RPS
    pltpu.sync_copy(z128_hbm.at[pl.ds(row0, RPS)], acc_sh.at[pl.ds(row0, RPS)])

    @pl.when(sid == NS - 1)
    def _init_tail():
        tl = pl.ds(N - RTAIL, RTAIL)
        pltpu.sync_copy(z128_hbm.at[tl], acc_sh.at[tl])

    def _zdd(i, c):
        sl = pl.ds(i * 16, 16)
        den_v[sl] = jnp.zeros((16,), jnp.float32)
        deg_v[sl] = jnp.zeros((16,), jnp.float32)
        return c
    lax.fori_loop(0, N // 16, _zdd, 0)

    plsc.subcore_barrier()

    ones16 = jnp.ones((16,), jnp.float32)
    ebase = wid * EPW

    def _chunk(j, carry):
        base = ebase + j * K
        pltpu.sync_copy(src_hbm.at[pl.ds(base, K)], src_v)
        pltpu.sync_copy(dst_hbm.at[pl.ds(base, K)], dst_v)
        pltpu.sync_copy(ew_hbm.at[pl.ds(base, K)], ew_v.at[pl.ds(0, K)])
        pltpu.async_copy(h_hbm.at[src_v], rows_v, sem).wait()

        def _row(i, c):
            wk = jnp.full((16,), ew_v[pl.ds(i, 16)][0], jnp.float32)
            for g in range(D // 16):
                sl = pl.ds(g * 16, 16)
                rows_v[i, sl] = rows_v[i, sl] * wk
            return c
        lax.fori_loop(0, K, _row, 0)

        def _dd(t, c):
            sl = pl.ds(t * 16, 16)
            iv = dst_v[sl]
            plsc.addupdate_scatter(den_v, [iv], ew_v[sl])
            plsc.addupdate_scatter(deg_v, [iv], ones16)
            return c
        lax.fori_loop(0, K // 16, _dd, 0)

        pltpu.sync_copy(rows_v, acc_sh.at[dst_v], add=True)
        return carry

    lax.fori_loop(0, NCHUNK, _chunk, 0)

    plsc.subcore_barrier()

    pltpu.sync_copy(acc_sh.at[pl.ds(row0, RPS)],
                    num_out.at[cid, pl.ds(row0, RPS)])

    @pl.when(sid == NS - 1)
    def _out_tail():
        tl = pl.ds(N - RTAIL, RTAIL)
        pltpu.sync_copy(acc_sh.at[tl], num_out.at[cid, tl])

    pltpu.sync_copy(den_v, den_out.at[wid])
    pltpu.sync_copy(deg_v, deg_out.at[wid])


@functools.lru_cache(maxsize=None)
def _get_sc_call():
    return pl.kernel(
        _sc_body,
        out_type=[jax.ShapeDtypeStruct((NC, N, D), jnp.float32),
                  jax.ShapeDtypeStruct((NW, N), jnp.float32),
                  jax.ShapeDtypeStruct((NW, N), jnp.float32)],
        mesh=plsc.VectorSubcoreMesh(core_axis_name="c", subcore_axis_name="s",
                                    num_cores=NC, num_subcores=NS),
        compiler_params=pltpu.CompilerParams(needs_layout_passes=False),
        scratch_types=[
            pltpu.VMEM((K,), jnp.int32),
            pltpu.VMEM((K,), jnp.int32),
            pltpu.VMEM((K + 16,), jnp.float32),
            pltpu.VMEM((K, D), jnp.float32),
            pltpu.VMEM((N,), jnp.float32),
            pltpu.VMEM((N,), jnp.float32),
            pltpu.VMEM_SHARED((N, D), jnp.float32),
            pltpu.SemaphoreType.DMA,
        ],
    )


def kernel(feat, edge_index, edge_w, Q_w, Q_b, W_w, W_b):
    feat = feat.astype(jnp.float32)
    src = edge_index[0].astype(jnp.int32)
    dst = edge_index[1].astype(jnp.int32)
    ew = edge_w.astype(jnp.float32)

    h = pl.pallas_call(
        _mm1_body,
        out_shape=jax.ShapeDtypeStruct((N, D), jnp.float32),
    )(feat, Q_w, Q_b.reshape(1, D))

    z128 = jnp.zeros((N, D), jnp.float32)
    num_p, den_p, deg_p = _get_sc_call()(h, src, dst, ew, z128)

    rst = pl.pallas_call(
        _post_body,
        out_shape=jax.ShapeDtypeStruct((N, D), jnp.float32),
    )(feat, num_p, den_p, deg_p, W_w[:D], W_w[D:], W_b.reshape(1, D))
    return rst


# speedup vs baseline: 8.2132x; 1.7578x over previous
import functools

import jax
import jax.numpy as jnp
from jax import lax
from jax.experimental import pallas as pl
from jax.experimental.pallas import tpu as pltpu
from jax.experimental.pallas import tpu_sc as plsc

N = 10000
E = 320000
D = 128
NC = 2
NS = 16
NW = NC * NS
EPW = E // NW
K = 80
NCHUNK = EPW // K
RPS = 624
RTAIL = N - RPS * NS


def _mm1_body(x_ref, w_ref, b_ref, o_ref):
    o_ref[...] = jnp.maximum(
        jnp.dot(x_ref[...], w_ref[...], preferred_element_type=jnp.float32)
        + b_ref[...], 0.0)


def _post_body(feat_ref, num_ref, den_ref, deg_ref, w1_ref, w2_ref, b_ref,
               o_ref):
    num = num_ref[0] + num_ref[1]
    den = jnp.sum(den_ref[...], axis=0)[:, None]
    deg = jnp.sum(deg_ref[...], axis=0)[:, None]
    safe_den = jnp.where(den > 0, den, 1.0)
    agg = jnp.where(den > 0, num / safe_den, 0.0)
    rst = jnp.dot(feat_ref[...], w1_ref[...], preferred_element_type=jnp.float32)
    rst = rst + jnp.dot(agg, w2_ref[...], preferred_element_type=jnp.float32)
    rst = jnp.maximum(rst + b_ref[...], 0.0)
    rst = rst * (1.0 / jnp.maximum(deg, 1.0))
    denom = jnp.sqrt(jnp.sum(rst * rst, axis=1, keepdims=True))
    safe_denom = jnp.where(denom > 0, denom, 1.0)
    o_ref[...] = jnp.where(denom > 0, rst / safe_denom, 0.0)


def _sc_body(h_hbm, src_hbm, dst_hbm, ew_hbm, z128_hbm,
             num_out, den_out, deg_out,
             src_v, dst_v, ew_v, rows_v, den_v, deg_v, acc_sh,
             sem_g0, sem_g1, sem_i0, sem_i1):
    cid = lax.axis_index("c")
    sid = lax.axis_index("s")
    wid = cid * NS + sid
    sem_g = (sem_g0, sem_g1)
    sem_i = (sem_i0, sem_i1)

    row0 = sid * RPS
    pltpu.sync_copy(z128_hbm.at[pl.ds(row0, RPS)], acc_sh.at[pl.ds(row0, RPS)])

    @pl.when(sid == NS - 1)
    def _init_tail():
        tl = pl.ds(N - RTAIL, RTAIL)
        pltpu.sync_copy(z128_hbm.at[tl], acc_sh.at[tl])

    def _zdd(i, c):
        sl = pl.ds(i * 16, 16)
        den_v[sl] = jnp.zeros((16,), jnp.float32)
        deg_v[sl] = jnp.zeros((16,), jnp.float32)
        return c
    lax.fori_loop(0, N // 16, _zdd, 0)

    plsc.subcore_barrier()

    ones16 = jnp.ones((16,), jnp.float32)
    ebase = wid * EPW

    def _load_idx(j, b, sem):
        base = ebase + j * K
        pltpu.async_copy(src_hbm.at[pl.ds(base, K)], src_v.at[b], sem)
        pltpu.async_copy(dst_hbm.at[pl.ds(base, K)], dst_v.at[b], sem)
        pltpu.async_copy(ew_hbm.at[pl.ds(base, K)], ew_v.at[b], sem)

    def _drain_idx(b, sem):
        pltpu.make_async_copy(src_hbm.at[pl.ds(0, K)], src_v.at[b], sem).wait()
        pltpu.make_async_copy(dst_hbm.at[pl.ds(0, K)], dst_v.at[b], sem).wait()
        pltpu.make_async_copy(ew_hbm.at[pl.ds(0, K)], ew_v.at[b], sem).wait()

    def _compute_scatter(b):
        def _blk(t, c):
            w16 = ew_v[b, pl.ds(t * 16, 16)]
            for l in range(16):
                wk = jnp.full((16,), w16[l], jnp.float32)
                i = t * 16 + l
                for g in range(D // 16):
                    sl = pl.ds(g * 16, 16)
                    rows_v[b, i, sl] = rows_v[b, i, sl] * wk
            return c
        lax.fori_loop(0, K // 16, _blk, 0)

        for t in range(K // 16):
            sl = pl.ds(t * 16, 16)
            iv = dst_v[b, sl]
            plsc.addupdate_scatter(den_v, [iv], ew_v[b, sl])
            plsc.addupdate_scatter(deg_v, [iv], ones16)

        pltpu.sync_copy(rows_v.at[b], acc_sh.at[dst_v.at[b]], add=True)

    _load_idx(0, 0, sem_i[0])
    _drain_idx(0, sem_i[0])
    _load_idx(1, 1, sem_i[1])
    _drain_idx(1, sem_i[1])

    def _pair(m, carry):
        @pl.when(m > 0)
        def _():
            _drain_idx(0, sem_i[0])
            _drain_idx(1, sem_i[1])

        d0 = pltpu.async_copy(h_hbm.at[src_v.at[0]], rows_v.at[0], sem_g[0])
        d1 = pltpu.async_copy(h_hbm.at[src_v.at[1]], rows_v.at[1], sem_g[1])
        d0.wait()
        _compute_scatter(0)

        @pl.when(2 * m + 2 < NCHUNK)
        def _():
            _load_idx(2 * m + 2, 0, sem_i[0])

        d1.wait()
        _compute_scatter(1)

        @pl.when(2 * m + 3 < NCHUNK)
        def _():
            _load_idx(2 * m + 3, 1, sem_i[1])
        return carry

    lax.fori_loop(0, NCHUNK // 2, _pair, 0)

    if NCHUNK % 2:
        _drain_idx(0, sem_i[0])
        pltpu.async_copy(h_hbm.at[src_v.at[0]], rows_v.at[0], sem_g[0]).wait()
        _compute_scatter(0)

    plsc.subcore_barrier()

    pltpu.sync_copy(acc_sh.at[pl.ds(row0, RPS)],
                    num_out.at[cid, pl.ds(row0, RPS)])

    @pl.when(sid == NS - 1)
    def _out_tail():
        tl = pl.ds(N - RTAIL, RTAIL)
        pltpu.sync_copy(acc_sh.at[tl], num_out.at[cid, tl])

    pltpu.sync_copy(den_v, den_out.at[wid])
    pltpu.sync_copy(deg_v, deg_out.at[wid])


@functools.lru_cache(maxsize=None)
def _get_sc_call():
    return pl.kernel(
        _sc_body,
        out_type=[jax.ShapeDtypeStruct((NC, N, D), jnp.float32),
                  jax.ShapeDtypeStruct((NW, N), jnp.float32),
                  jax.ShapeDtypeStruct((NW, N), jnp.float32)],
        mesh=plsc.VectorSubcoreMesh(core_axis_name="c", subcore_axis_name="s",
                                    num_cores=NC, num_subcores=NS),
        compiler_params=pltpu.CompilerParams(needs_layout_passes=False),
        scratch_types=[
            pltpu.VMEM((2, K), jnp.int32),
            pltpu.VMEM((2, K), jnp.int32),
            pltpu.VMEM((2, K), jnp.float32),
            pltpu.VMEM((2, K, D), jnp.float32),
            pltpu.VMEM((N,), jnp.float32),
            pltpu.VMEM((N,), jnp.float32),
            pltpu.VMEM_SHARED((N, D), jnp.float32),
            pltpu.SemaphoreType.DMA,
            pltpu.SemaphoreType.DMA,
            pltpu.SemaphoreType.DMA,
            pltpu.SemaphoreType.DMA,
        ],
    )


def kernel(feat, edge_index, edge_w, Q_w, Q_b, W_w, W_b):
    feat = feat.astype(jnp.float32)
    src = edge_index[0].astype(jnp.int32)
    dst = edge_index[1].astype(jnp.int32)
    ew = edge_w.astype(jnp.float32)

    h = pl.pallas_call(
        _mm1_body,
        out_shape=jax.ShapeDtypeStruct((N, D), jnp.float32),
    )(feat, Q_w, Q_b.reshape(1, D))

    z128 = jnp.zeros((N, D), jnp.float32)
    num_p, den_p, deg_p = _get_sc_call()(h, src, dst, ew, z128)

    rst = pl.pallas_call(
        _post_body,
        out_shape=jax.ShapeDtypeStruct((N, D), jnp.float32),
    )(feat, num_p, den_p, deg_p, W_w[:D], W_w[D:], W_b.reshape(1, D))
    return rst
